# baseline (speedup 1.0000x reference)
import functools

import jax
import jax.numpy as jnp
import numpy as np
from jax import lax
from jax.experimental import pallas as pl
from jax.experimental.pallas import tpu as pltpu
from jax.experimental.pallas import tpu_sc as plsc

_N, _E, _D, _G = 10000, 320000, 128, 64
_NC, _NS = 2, 16
_NW = _NC * _NS
_EPW = _E // _NW
_C = 40
_NCH = _EPW // _C
_RS0 = 624
_RST = _N - (_NS - 1) * _RS0 - _RS0
_BN = 1000
_NB = _N // _BN
_PERM = np.array(sum([[32 * j + 2 * t for t in range(16)]
                      + [32 * j + 2 * t + 1 for t in range(16)]
                      for j in range(_D // 32)], []), dtype=np.int32)


def _qkvr(x, Wk, bk, Wq, bq, Wv, bv, Ws, bc):
    def body(x_ref, wk, bk_r, wq, bq_r, wv, bv_r, ws, bc_r,
             kqv_ref, r_ref):
        xb = x_ref[...]
        kk = (jnp.dot(xb, wk[...], preferred_element_type=jnp.float32)
              + bk_r[...]).astype(jnp.bfloat16)
        kqv_ref[0, :, 0:_D] = kk
        kqv_ref[0, :, _D:2 * _D] = kk
        kqv_ref[1, :, 0:_D] = (
            jnp.dot(xb, wq[...], preferred_element_type=jnp.float32)
            + bq_r[...]).astype(jnp.bfloat16)
        kqv_ref[1, :, _D:2 * _D] = (
            jnp.dot(xb, wv[...], preferred_element_type=jnp.float32)
            + bv_r[...]).astype(jnp.bfloat16)
        r_ref[...] = jnp.dot(xb, ws[...], preferred_element_type=jnp.float32) + bc_r[...]

    row = pl.BlockSpec((_BN, _D), lambda i: (i, 0))
    full = pl.BlockSpec((_D, _D), lambda i: (0, 0))
    vec = pl.BlockSpec((1, _D), lambda i: (0, 0))
    return pl.pallas_call(
        body,
        grid=(_NB,),
        in_specs=[row, full, vec, full, vec, full, vec, full, vec],
        out_specs=[pl.BlockSpec((2, _BN, 2 * _D), lambda i: (0, i, 0)), row],
        out_shape=[jax.ShapeDtypeStruct((2, _N, 2 * _D), jnp.bfloat16),
                   jax.ShapeDtypeStruct((_N, _D), jnp.float32)],
    )(x, Wk, bk, Wq, bq, Wv, bv, Ws, bc)


def _edge_sc(kqv, gidx, zeros):
    mesh = plsc.VectorSubcoreMesh(core_axis_name="c", subcore_axis_name="s",
                                  num_cores=_NC, num_subcores=_NS)

    @functools.partial(
        pl.kernel,
        out_type=jax.ShapeDtypeStruct((_NC, _N, _D), jnp.float32),
        mesh=mesh,
        scratch_types=[
            pltpu.VMEM((4, 2 * _C), jnp.int32),
            pltpu.VMEM((4, _C), jnp.int32),
            pltpu.VMEM((2, 2 * _C, _D), jnp.int32),
            pltpu.VMEM((2, _C, _D), jnp.float32),
            pltpu.VMEM_SHARED((_N, _D), jnp.float32),
            pltpu.SemaphoreType.DMA,
            pltpu.SemaphoreType.DMA,
            pltpu.SemaphoreType.DMA,
            pltpu.SemaphoreType.DMA,
            pltpu.SemaphoreType.DMA,
        ],
    )
    def body(kqv_hbm, gidx_hbm, z_hbm, out_hbm,
             gix, six, kqvb, msgb, agg, gs0, gs1, ss0, ss1, isem):
        gsem = (gs0, gs1)
        ssem = (ss0, ss1)
        c = lax.axis_index("c")
        s = lax.axis_index("s")
        wid = c * _NS + s
        row0 = pl.multiple_of(s * _RS0, 8)
        pltpu.sync_copy(z_hbm.at[pl.ds(row0, _RS0)], agg.at[pl.ds(row0, _RS0)])

        @pl.when(s == _NS - 1)
        def _tail_init():
            pltpu.sync_copy(z_hbm.at[pl.ds(_NS * _RS0, _RST)],
                            agg.at[pl.ds(_NS * _RS0, _RST)])

        plsc.subcore_barrier()

        def issue_idx(t, islot):
            pltpu.async_copy(gidx_hbm.at[wid, t], gix.at[islot], isem)
            pltpu.async_copy(gidx_hbm.at[wid, t, pl.ds(0, _C)],
                             six.at[islot], isem)

        def wait_idx(islot):
            pltpu.make_async_copy(gidx_hbm.at[0, 0], gix.at[islot],
                                  isem).wait()
            pltpu.make_async_copy(gidx_hbm.at[0, 0, pl.ds(0, _C)],
                                  six.at[islot], isem).wait()

        def issue_gather(t, islot, b):
            pltpu.async_copy(kqv_hbm.at[gix.at[islot]], kqvb.at[b], gsem[b])

        def drain_gather(b):
            pltpu.make_async_copy(kqv_hbm.at[pl.ds(0, 2 * _C)], kqvb.at[b],
                                  gsem[b]).wait()

        def drain_scatter(b):
            pltpu.make_async_copy(z_hbm.at[pl.ds(0, _C)], msgb.at[b],
                                  ssem[b]).wait()

        def compute(islot, b):
            m16 = jnp.int32(-65536)

            @plsc.parallel_loop(0, _C)
            def row(rr):
                for j in range(_D // 32):
                    sl = pl.ds(j * 16, 16)
                    kw = kqvb[b, rr, sl]
                    qw = kqvb[b, _C + rr, sl]
                    vw = kqvb[b, _C + rr, pl.ds(_D // 2 + j * 16, 16)]
                    klo = lax.bitcast_convert_type(kw << 16, jnp.float32)
                    khi = lax.bitcast_convert_type(kw & m16, jnp.float32)
                    qlo = lax.bitcast_convert_type(qw << 16, jnp.float32)
                    qhi = lax.bitcast_convert_type(qw & m16, jnp.float32)
                    vlo = lax.bitcast_convert_type(vw << 16, jnp.float32)
                    vhi = lax.bitcast_convert_type(vw & m16, jnp.float32)
                    msgb[b, rr, pl.ds(j * 32, 16)] = (
                        vlo / (1.0 + jnp.exp(-(klo + qlo))))
                    msgb[b, rr, pl.ds(j * 32 + 16, 16)] = (
                        vhi / (1.0 + jnp.exp(-(khi + qhi))))

            pltpu.async_copy(msgb.at[b], agg.at[six.at[islot]], ssem[b],
                             add=True)

        pltpu.sync_copy(gidx_hbm.at[wid, 0], gix.at[0])
        pltpu.sync_copy(gidx_hbm.at[wid, 0, pl.ds(0, _C)], six.at[0])
        issue_gather(0, 0, 0)
        issue_idx(1, 1)

        def quad(t4, carry):
            for u in range(4):
                t = 4 * t4 + u
                b = u % 2
                nb = 1 - b
                if u == 0:
                    @pl.when(t4 > 0)
                    def _():
                        drain_scatter(nb)
                else:
                    drain_scatter(nb)
                wait_idx((u + 1) % 4)
                issue_gather(t + 1, (u + 1) % 4, nb)
                issue_idx(t + 2, (u + 2) % 4)
                drain_gather(b)
                compute(u % 4, b)
            return carry

        lax.fori_loop(0, (_NCH - 2) // 4, quad, 0)
        drain_scatter(1)
        wait_idx(1)
        issue_gather(_NCH - 1, 1, 1)
        drain_gather(0)
        compute(0, 0)
        drain_gather(1)
        compute(1, 1)
        drain_scatter(0)
        drain_scatter(1)
        plsc.subcore_barrier()
        pltpu.sync_copy(agg.at[pl.ds(row0, _RS0)],
                        out_hbm.at[c, pl.ds(row0, _RS0)])

        @pl.when(s == _NS - 1)
        def _tail_out():
            pltpu.sync_copy(agg.at[pl.ds(_NS * _RS0, _RST)],
                            out_hbm.at[c, pl.ds(_NS * _RS0, _RST)])

    return body(kqv, gidx, zeros)


def _finish(agg, r, batch3, gamma, beta, Wout, bout):
    def body(agg_ref, r_ref, b_ref, g_ref, be_ref, wo_ref, bo_ref, out_ref,
             sum_s, sq_s, seg_s, cnt_s):
        i = pl.program_id(0)

        @pl.when(i == 0)
        def _init():
            sum_s[...] = jnp.zeros_like(sum_s)
            sq_s[...] = jnp.zeros_like(sq_s)
            seg_s[...] = jnp.zeros_like(seg_s)
            cnt_s[...] = jnp.zeros_like(cnt_s)

        h = agg_ref[0] + agg_ref[1] + r_ref[...]
        h = jnp.maximum(h, 0.0)
        sum_s[...] += jnp.sum(h, axis=0, keepdims=True)
        sq_s[...] += jnp.sum(h * h, axis=0, keepdims=True)
        b = b_ref[0]
        gids = lax.broadcasted_iota(jnp.int32, (_G, _BN), 0)
        ohT = (gids == b).astype(jnp.float32)
        seg_s[...] += lax.dot_general(ohT, h, (((1,), (0,)), ((), ())),
                                      preferred_element_type=jnp.float32)
        cnt_s[...] += jnp.sum(ohT, axis=1, keepdims=True)

        @pl.when(i == _NB - 1)
        def _fin():
            mean = sum_s[...] / _N
            var = sq_s[...] / _N - mean * mean
            inv = lax.rsqrt(var + 1e-5)
            pooled = ((seg_s[...] - cnt_s[...] * mean) * (inv * g_ref[...])
                      + cnt_s[...] * be_ref[...])
            out_ref[...] = jnp.dot(pooled, wo_ref[...],
                                   preferred_element_type=jnp.float32) + bo_ref[...]

    return pl.pallas_call(
        body,
        grid=(_NB,),
        in_specs=[
            pl.BlockSpec((_NC, _BN, _D), lambda i: (0, i, 0)),
            pl.BlockSpec((_BN, _D), lambda i: (i, 0)),
            pl.BlockSpec((1, 1, _BN), lambda i: (i, 0, 0)),
            pl.BlockSpec((1, _D), lambda i: (0, 0)),
            pl.BlockSpec((1, _D), lambda i: (0, 0)),
            pl.BlockSpec((_D, 1), lambda i: (0, 0)),
            pl.BlockSpec((1, 1), lambda i: (0, 0)),
        ],
        out_specs=pl.BlockSpec((_G, 1), lambda i: (0, 0)),
        out_shape=jax.ShapeDtypeStruct((_G, 1), jnp.float32),
        scratch_shapes=[
            pltpu.VMEM((1, _D), jnp.float32),
            pltpu.VMEM((1, _D), jnp.float32),
            pltpu.VMEM((_G, _D), jnp.float32),
            pltpu.VMEM((_G, 1), jnp.float32),
        ],
    )(agg, r, batch3, gamma, beta, Wout, bout)


def kernel(x, edge_index, batch, Wk, bk, Wq, bq, Wv, bv, Ws, b_conv,
           bn_gamma, bn_beta, W_out, b_out):
    s3 = edge_index[0].reshape(_NW, _NCH, 1, _C)
    d3 = edge_index[1].reshape(_NW, _NCH, 1, _C)
    gidx = jnp.concatenate([d3, s3 + _N], axis=2).reshape(_NW, _NCH, 2 * _C)
    kqv, r = _qkvr(x, Wk, bk.reshape(1, _D), Wq, bq.reshape(1, _D),
                   Wv, bv.reshape(1, _D), Ws[:, _PERM],
                   b_conv[_PERM].reshape(1, _D))
    kqv_i32 = jax.lax.bitcast_convert_type(
        kqv.reshape(2 * _N, _D, 2), jnp.int32)
    zeros = jnp.zeros((_N, _D), jnp.float32)
    agg = _edge_sc(kqv_i32, gidx, zeros)
    batch3 = batch.reshape(_NB, 1, _BN)
    return _finish(agg, r, batch3, bn_gamma[_PERM].reshape(1, _D),
                   bn_beta[_PERM].reshape(1, _D), W_out[_PERM, :],
                   b_out.reshape(1, 1))


# speedup vs baseline: 1.4134x; 1.4134x over previous
import functools

import jax
import jax.numpy as jnp
from jax import lax
from jax.experimental import pallas as pl
from jax.experimental.pallas import tpu as pltpu
from jax.experimental.pallas import tpu_sc as plsc

_N, _E, _D, _G = 10000, 320000, 128, 64
_NC, _NS = 2, 16
_NW = _NC * _NS
_EPW = _E // _NW
_C = 40
_NCH = _EPW // _C
_RS0 = 624
_RST = _N - (_NS - 1) * _RS0 - _RS0
_BN = 1000
_NB = _N // _BN


def _qkvr(x, Wk, bk, Wq, bq, Wv, bv, Ws, bc):
    def body(x_ref, wk, bk_r, wq, bq_r, wv, bv_r, ws, bc_r,
             kqv_ref, r_ref):
        xb = x_ref[...]
        kqv_ref[0] = jnp.dot(xb, wk[...], preferred_element_type=jnp.float32) + bk_r[...]
        kqv_ref[1] = jnp.dot(xb, wq[...], preferred_element_type=jnp.float32) + bq_r[...]
        kqv_ref[2] = jnp.dot(xb, wv[...], preferred_element_type=jnp.float32) + bv_r[...]
        r_ref[...] = jnp.dot(xb, ws[...], preferred_element_type=jnp.float32) + bc_r[...]

    row = pl.BlockSpec((_BN, _D), lambda i: (i, 0))
    full = pl.BlockSpec((_D, _D), lambda i: (0, 0))
    vec = pl.BlockSpec((1, _D), lambda i: (0, 0))
    return pl.pallas_call(
        body,
        grid=(_NB,),
        in_specs=[row, full, vec, full, vec, full, vec, full, vec],
        out_specs=[pl.BlockSpec((3, _BN, _D), lambda i: (0, i, 0)), row],
        out_shape=[jax.ShapeDtypeStruct((3, _N, _D), jnp.float32),
                   jax.ShapeDtypeStruct((_N, _D), jnp.float32)],
    )(x, Wk, bk, Wq, bq, Wv, bv, Ws, bc)


def _edge_sc(kqv, gidx, zeros):
    mesh = plsc.VectorSubcoreMesh(core_axis_name="c", subcore_axis_name="s",
                                  num_cores=_NC, num_subcores=_NS)

    @functools.partial(
        pl.kernel,
        out_type=jax.ShapeDtypeStruct((_NC, _N, _D), jnp.float32),
        mesh=mesh,
        scratch_types=[
            pltpu.VMEM((4, 3 * _C), jnp.int32),
            pltpu.VMEM((4, _C), jnp.int32),
            pltpu.VMEM((2, 3 * _C, _D), jnp.float32),
            pltpu.VMEM_SHARED((_N, _D), jnp.float32),
            pltpu.SemaphoreType.DMA,
            pltpu.SemaphoreType.DMA,
            pltpu.SemaphoreType.DMA,
            pltpu.SemaphoreType.DMA,
            pltpu.SemaphoreType.DMA,
        ],
    )
    def body(kqv_hbm, gidx_hbm, z_hbm, out_hbm,
             gix, six, kqvb, agg, gs0, gs1, ss0, ss1, isem):
        gsem = (gs0, gs1)
        ssem = (ss0, ss1)
        c = lax.axis_index("c")
        s = lax.axis_index("s")
        wid = c * _NS + s
        row0 = pl.multiple_of(s * _RS0, 8)
        pltpu.sync_copy(z_hbm.at[pl.ds(row0, _RS0)], agg.at[pl.ds(row0, _RS0)])

        @pl.when(s == _NS - 1)
        def _tail_init():
            pltpu.sync_copy(z_hbm.at[pl.ds(_NS * _RS0, _RST)],
                            agg.at[pl.ds(_NS * _RS0, _RST)])

        plsc.subcore_barrier()

        def issue_idx(t, islot):
            pltpu.async_copy(gidx_hbm.at[wid, t], gix.at[islot], isem)
            pltpu.async_copy(gidx_hbm.at[wid, t, pl.ds(0, _C)],
                             six.at[islot], isem)

        def wait_idx(islot):
            pltpu.make_async_copy(gidx_hbm.at[0, 0], gix.at[islot],
                                  isem).wait()
            pltpu.make_async_copy(gidx_hbm.at[0, 0, pl.ds(0, _C)],
                                  six.at[islot], isem).wait()

        def issue_gather(t, islot, b):
            pltpu.async_copy(kqv_hbm.at[gix.at[islot]], kqvb.at[b], gsem[b])

        def drain_gather(b):
            pltpu.make_async_copy(kqv_hbm.at[pl.ds(0, 3 * _C)], kqvb.at[b],
                                  gsem[b]).wait()

        def drain_scatter(b):
            pltpu.make_async_copy(kqv_hbm.at[pl.ds(0, _C)],
                                  kqvb.at[b, pl.ds(2 * _C, _C)],
                                  ssem[b]).wait()

        def compute(islot, b):
            @plsc.parallel_loop(0, _C)
            def row(rr):
                for j in range(_D // 16):
                    sl = pl.ds(j * 16, 16)
                    z = kqvb[b, rr, sl] + kqvb[b, _C + rr, sl]
                    kqvb[b, 2 * _C + rr, sl] = (kqvb[b, 2 * _C + rr, sl]
                                                / (1.0 + jnp.exp(-z)))

            pltpu.async_copy(kqvb.at[b, pl.ds(2 * _C, _C)],
                             agg.at[six.at[islot]], ssem[b], add=True)

        pltpu.sync_copy(gidx_hbm.at[wid, 0], gix.at[0])
        pltpu.sync_copy(gidx_hbm.at[wid, 0, pl.ds(0, _C)], six.at[0])
        issue_gather(0, 0, 0)
        issue_idx(1, 1)

        def quad(t4, carry):
            for u in range(4):
                t = 4 * t4 + u
                b = u % 2
                nb = 1 - b
                if u == 0:
                    @pl.when(t4 > 0)
                    def _():
                        drain_scatter(nb)
                else:
                    drain_scatter(nb)
                wait_idx((u + 1) % 4)
                issue_gather(t + 1, (u + 1) % 4, nb)
                issue_idx(t + 2, (u + 2) % 4)
                drain_gather(b)
                compute(u % 4, b)
            return carry

        lax.fori_loop(0, (_NCH - 2) // 4, quad, 0)
        drain_scatter(1)
        wait_idx(1)
        issue_gather(_NCH - 1, 1, 1)
        drain_gather(0)
        compute(0, 0)
        drain_gather(1)
        compute(1, 1)
        drain_scatter(0)
        drain_scatter(1)
        plsc.subcore_barrier()
        pltpu.sync_copy(agg.at[pl.ds(row0, _RS0)],
                        out_hbm.at[c, pl.ds(row0, _RS0)])

        @pl.when(s == _NS - 1)
        def _tail_out():
            pltpu.sync_copy(agg.at[pl.ds(_NS * _RS0, _RST)],
                            out_hbm.at[c, pl.ds(_NS * _RS0, _RST)])

    return body(kqv, gidx, zeros)


def _finish(agg, r, batch3, gamma, beta, Wout, bout):
    def body(agg_ref, r_ref, b_ref, g_ref, be_ref, wo_ref, bo_ref, out_ref,
             sum_s, sq_s, seg_s, cnt_s):
        i = pl.program_id(0)

        @pl.when(i == 0)
        def _init():
            sum_s[...] = jnp.zeros_like(sum_s)
            sq_s[...] = jnp.zeros_like(sq_s)
            seg_s[...] = jnp.zeros_like(seg_s)
            cnt_s[...] = jnp.zeros_like(cnt_s)

        h = agg_ref[0] + agg_ref[1] + r_ref[...]
        h = jnp.maximum(h, 0.0)
        sum_s[...] += jnp.sum(h, axis=0, keepdims=True)
        sq_s[...] += jnp.sum(h * h, axis=0, keepdims=True)
        b = b_ref[0]
        gids = lax.broadcasted_iota(jnp.int32, (_G, _BN), 0)
        ohT = (gids == b).astype(jnp.float32)
        seg_s[...] += lax.dot_general(ohT, h, (((1,), (0,)), ((), ())),
                                      preferred_element_type=jnp.float32)
        cnt_s[...] += jnp.sum(ohT, axis=1, keepdims=True)

        @pl.when(i == _NB - 1)
        def _fin():
            mean = sum_s[...] / _N
            var = sq_s[...] / _N - mean * mean
            inv = lax.rsqrt(var + 1e-5)
            pooled = ((seg_s[...] - cnt_s[...] * mean) * (inv * g_ref[...])
                      + cnt_s[...] * be_ref[...])
            out_ref[...] = jnp.dot(pooled, wo_ref[...],
                                   preferred_element_type=jnp.float32) + bo_ref[...]

    return pl.pallas_call(
        body,
        grid=(_NB,),
        in_specs=[
            pl.BlockSpec((_NC, _BN, _D), lambda i: (0, i, 0)),
            pl.BlockSpec((_BN, _D), lambda i: (i, 0)),
            pl.BlockSpec((1, 1, _BN), lambda i: (i, 0, 0)),
            pl.BlockSpec((1, _D), lambda i: (0, 0)),
            pl.BlockSpec((1, _D), lambda i: (0, 0)),
            pl.BlockSpec((_D, 1), lambda i: (0, 0)),
            pl.BlockSpec((1, 1), lambda i: (0, 0)),
        ],
        out_specs=pl.BlockSpec((_G, 1), lambda i: (0, 0)),
        out_shape=jax.ShapeDtypeStruct((_G, 1), jnp.float32),
        scratch_shapes=[
            pltpu.VMEM((1, _D), jnp.float32),
            pltpu.VMEM((1, _D), jnp.float32),
            pltpu.VMEM((_G, _D), jnp.float32),
            pltpu.VMEM((_G, 1), jnp.float32),
        ],
    )(agg, r, batch3, gamma, beta, Wout, bout)


def kernel(x, edge_index, batch, Wk, bk, Wq, bq, Wv, bv, Ws, b_conv,
           bn_gamma, bn_beta, W_out, b_out):
    s3 = edge_index[0].reshape(_NW, _NCH, 1, _C)
    d3 = edge_index[1].reshape(_NW, _NCH, 1, _C)
    gidx = jnp.concatenate([d3, s3 + _N, s3 + 2 * _N],
                           axis=2).reshape(_NW, _NCH, 3 * _C)
    kqv, r = _qkvr(x, Wk, bk.reshape(1, _D), Wq, bq.reshape(1, _D),
                   Wv, bv.reshape(1, _D), Ws, b_conv.reshape(1, _D))
    zeros = jnp.zeros((_N, _D), jnp.float32)
    agg = _edge_sc(kqv.reshape(3 * _N, _D), gidx, zeros)
    batch3 = batch.reshape(_NB, 1, _BN)
    return _finish(agg, r, batch3, bn_gamma.reshape(1, _D),
                   bn_beta.reshape(1, _D), W_out, b_out.reshape(1, 1))
